# baseline (speedup 1.0000x reference)
import functools

import jax
import jax.numpy as jnp
from jax import lax
from jax.experimental import pallas as pl
from jax.experimental.pallas import tpu as pltpu
from jax.experimental.pallas import tpu_sc as plsc

N = 2048
NW = 32
BANDS_PER_W = 8
BIG_I32 = 2 ** 30


def _scan_body(g_hbm, cand_hbm, candi_hbm,
               ddiag_v, rowbuf, stage, stagei, dtiles, dloc, spdiag,
               rsem0, rsem1, dtsem):
    cid = lax.axis_index("c")
    sid = lax.axis_index("s")
    wid = sid * 2 + cid

    lane = lax.iota(jnp.int32, 16)
    rsems = (rsem0, rsem1)

    def start_band(u, buf):
        row0 = pl.multiple_of(8 * (wid + NW * u), 8)
        c0 = 256 * u
        return pltpu.async_copy(
            g_hbm.at[pl.ds(row0, 8), pl.ds(c0, N - c0)],
            rowbuf.at[buf, slice(None), pl.ds(c0, N - c0)],
            rsems[buf])

    band0_desc = start_band(0, 0)

    col0 = pl.multiple_of(128 * sid, 128)
    dt_descs = []
    for tt in range(16):
        row_off = pl.multiple_of(128 * sid + 8 * tt, 8)
        dt_descs.append(pltpu.async_copy(
            g_hbm.at[pl.ds(row_off, 8), pl.ds(col0, 128)],
            dtiles.at[tt], dtsem))
    for d in dt_descs:
        d.wait()
    for k in range(8):
        acc = jnp.zeros((16,), jnp.float32)
        for e in range(16):
            eidx = 16 * k + e
            tt, r = eidx // 8, eidx % 8
            c = 8 * tt + r
            chunk = dtiles[tt, r, pl.ds((c // 16) * 16, 16)]
            acc = jnp.where(lane == e, chunk[c % 16], acc)
        dloc[pl.ds(16 * k, 16)] = acc
    pltpu.sync_copy(dloc, spdiag.at[pl.ds(128 * sid, 128)])
    plsc.subcore_barrier()
    pltpu.sync_copy(spdiag, ddiag_v)

    w_even = (wid & 1) == 0
    bvs = tuple(jnp.full((16,), jnp.inf, jnp.float32) for _ in range(8))
    bis = tuple(jnp.full((16,), BIG_I32, jnp.int32) for _ in range(8))

    descs = [band0_desc, None]
    for u in range(BANDS_PER_W):
        buf = u % 2
        if u + 1 < BANDS_PER_W:
            descs[1 - buf] = start_band(u + 1, 1 - buf)
        descs[buf].wait()
        i0 = 8 * (wid + NW * u)
        dchunk = ddiag_v[pl.ds((i0 >> 4) * 16, 16)]
        a16s = []
        for r in range(8):
            a_s = jnp.where(w_even, dchunk[r], dchunk[r + 8])
            a16s.append(jnp.full((16,), a_s, jnp.float32))

        def v_body(v, carry, buf=buf, i0=i0, a16s=a16s):
            bvs, bis = carry
            b16 = ddiag_v[pl.ds(v * 16, 16)]
            j16 = v * 16 + lane
            nbvs, nbis = [], []
            for r in range(8):
                i = i0 + r
                a16 = a16s[r]
                c16 = rowbuf[buf, r, pl.ds(v * 16, 16)]
                t1 = b16 - c16
                den = a16 + b16 - 2.0 * c16 + 1e-8
                gam = t1 / den
                cr = b16 + gam * (c16 - b16)
                cost = jnp.where(c16 < b16, cr, b16)
                cost = jnp.where(c16 < a16, cost, a16)
                better = (cost < bvs[r]) & (j16 > i)
                nbvs.append(jnp.where(better, cost, bvs[r]))
                nbis.append(jnp.where(better, i * N + j16, bis[r]))
            return tuple(nbvs), tuple(nbis)

        bvs, bis = lax.fori_loop(16 * u, N // 16, v_body, (bvs, bis),
                                 unroll=2)

    for r in range(8):
        stage[r, pl.ds(0, 16)] = bvs[r]
        stagei[r, pl.ds(0, 16)] = bis[r]
    pltpu.sync_copy(stage, cand_hbm.at[wid])
    pltpu.sync_copy(stagei, candi_hbm.at[wid])


_scan = functools.partial(
    pl.kernel,
    out_type=(jax.ShapeDtypeStruct((NW, 8, 128), jnp.float32),
              jax.ShapeDtypeStruct((NW, 8, 128), jnp.int32)),
    mesh=plsc.VectorSubcoreMesh(core_axis_name="c", subcore_axis_name="s"),
    compiler_params=pltpu.CompilerParams(use_tc_tiling_on_sc=True),
    scratch_types=[
        pltpu.VMEM((N,), jnp.float32),
        pltpu.VMEM((2, 8, N), jnp.float32),
        pltpu.VMEM((8, 128), jnp.float32),
        pltpu.VMEM((8, 128), jnp.int32),
        pltpu.VMEM((16, 8, 128), jnp.float32),
        pltpu.VMEM((128,), jnp.float32),
        pltpu.VMEM_SHARED((N,), jnp.float32),
        pltpu.SemaphoreType.DMA,
        pltpu.SemaphoreType.DMA,
        pltpu.SemaphoreType.DMA,
    ],
)(_scan_body)


def _merge_body(cand_ref, candi_ref, g_ref, out_ref, cbuf, abuf, bbuf, msem):
    v = cand_ref[:, :, :16]
    ix = candi_ref[:, :, :16]
    m = jnp.min(v)
    win = jnp.min(jnp.where(v == m, ix, BIG_I32))
    i_min = win >> 11
    j_min = win & (N - 1)
    ja = pl.multiple_of((j_min >> 7) * 128, 128)
    ia = pl.multiple_of((i_min >> 7) * 128, 128)
    cp = pltpu.make_async_copy(
        g_ref.at[pl.ds(i_min, 1), pl.ds(ja, 128)], cbuf, msem)
    ap = pltpu.make_async_copy(
        g_ref.at[pl.ds(i_min, 1), pl.ds(ia, 128)], abuf, msem)
    bp = pltpu.make_async_copy(
        g_ref.at[pl.ds(j_min, 1), pl.ds(ja, 128)], bbuf, msem)
    cp.start()
    ap.start()
    bp.start()
    cp.wait()
    ap.wait()
    bp.wait()
    l8 = lax.broadcasted_iota(jnp.int32, (1, 128), 1)
    c = jnp.sum(jnp.where(l8 == (j_min & 127), cbuf[...], 0.0))
    a = jnp.sum(jnp.where(l8 == (i_min & 127), abuf[...], 0.0))
    b = jnp.sum(jnp.where(l8 == (j_min & 127), bbuf[...], 0.0))
    gw = (b - c) / (a + b - 2.0 * c + 1e-8)
    gw = jnp.where(c < b, gw, 0.0)
    gw = jnp.where(c < a, gw, 1.0)
    flat = (lax.broadcasted_iota(jnp.int32, (16, 128), 0) * 128
            + lax.broadcasted_iota(jnp.int32, (16, 128), 1))
    out_ref[...] = (jnp.where(flat == i_min, gw, 0.0)
                    + jnp.where(flat == j_min, 1.0 - gw, 0.0))


_merge = pl.pallas_call(
    _merge_body,
    in_specs=[
        pl.BlockSpec((NW, 8, 128), lambda: (0, 0, 0)),
        pl.BlockSpec((NW, 8, 128), lambda: (0, 0, 0)),
        pl.BlockSpec(memory_space=pl.ANY),
    ],
    out_shape=jax.ShapeDtypeStruct((16, 128), jnp.float32),
    scratch_shapes=[
        pltpu.VMEM((1, 128), jnp.float32),
        pltpu.VMEM((1, 128), jnp.float32),
        pltpu.VMEM((1, 128), jnp.float32),
        pltpu.SemaphoreType.DMA,
    ],
)


def kernel(grammian):
    cand, candi = _scan(grammian)
    return _merge(cand, candi, grammian).reshape(N)


# speedup vs baseline: 1.1651x; 1.1651x over previous
import functools

import jax
import jax.numpy as jnp
from jax import lax
from jax.experimental import pallas as pl
from jax.experimental.pallas import tpu as pltpu
from jax.experimental.pallas import tpu_sc as plsc

N = 2048
NW = 32
BANDS_PER_W = 8
BIG_I32 = 2 ** 30


def _scan_body(g_hbm, cand_hbm, candi_hbm,
               ddiag_v, rowbuf, stage, stagei, dtiles, dloc, spdiag,
               rsem0, rsem1, dtsem):
    cid = lax.axis_index("c")
    sid = lax.axis_index("s")
    wid = sid * 2 + cid

    lane = lax.iota(jnp.int32, 16)
    rsems = (rsem0, rsem1)

    def start_band(u, buf):
        row0 = pl.multiple_of(8 * (wid + NW * u), 8)
        c0 = 256 * u
        return pltpu.async_copy(
            g_hbm.at[pl.ds(row0, 8), pl.ds(c0, N - c0)],
            rowbuf.at[buf, slice(None), pl.ds(c0, N - c0)],
            rsems[buf])

    band0_desc = start_band(0, 0)

    col0 = pl.multiple_of(128 * sid, 128)
    dt_descs = []
    for tt in range(16):
        row_off = pl.multiple_of(128 * sid + 8 * tt, 8)
        dt_descs.append(pltpu.async_copy(
            g_hbm.at[pl.ds(row_off, 8), pl.ds(col0, 128)],
            dtiles.at[tt], dtsem))
    for d in dt_descs:
        d.wait()
    for k in range(8):
        acc = jnp.zeros((16,), jnp.float32)
        for e in range(16):
            eidx = 16 * k + e
            tt, r = eidx // 8, eidx % 8
            c = 8 * tt + r
            chunk = dtiles[tt, r, pl.ds((c // 16) * 16, 16)]
            acc = jnp.where(lane == e, chunk[c % 16], acc)
        dloc[pl.ds(16 * k, 16)] = acc
    pltpu.sync_copy(dloc, spdiag.at[pl.ds(128 * sid, 128)])
    plsc.subcore_barrier()
    pltpu.sync_copy(spdiag, ddiag_v)

    w_even = (wid & 1) == 0
    bvs = tuple(jnp.full((16,), jnp.inf, jnp.float32) for _ in range(8))
    bis = tuple(jnp.full((16,), BIG_I32, jnp.int32) for _ in range(8))

    descs = [band0_desc, None]
    for u in range(BANDS_PER_W):
        buf = u % 2
        if u + 1 < BANDS_PER_W:
            descs[1 - buf] = start_band(u + 1, 1 - buf)
        descs[buf].wait()
        i0 = 8 * (wid + NW * u)
        dchunk = ddiag_v[pl.ds((i0 >> 4) * 16, 16)]
        a16s = []
        for r in range(8):
            a_s = jnp.where(w_even, dchunk[r], dchunk[r + 8])
            a16s.append(jnp.full((16,), a_s, jnp.float32))

        def v_body(v, carry, buf=buf, i0=i0, a16s=a16s):
            bvs, bis = carry
            b16 = ddiag_v[pl.ds(v * 16, 16)]
            j16 = v * 16 + lane
            nbvs, nbis = [], []
            for r in range(8):
                i = i0 + r
                a16 = a16s[r]
                c16 = rowbuf[buf, r, pl.ds(v * 16, 16)]
                t1 = b16 - c16
                den = a16 + b16 - 2.0 * c16 + 1e-8
                gam = t1 / den
                cr = b16 + gam * (c16 - b16)
                cost = jnp.where(c16 < b16, cr, b16)
                cost = jnp.where(c16 < a16, cost, a16)
                better = (cost < bvs[r]) & (j16 > i)
                nbvs.append(jnp.where(better, cost, bvs[r]))
                nbis.append(jnp.where(better, i * N + j16, bis[r]))
            return tuple(nbvs), tuple(nbis)

        bvs, bis = lax.fori_loop(16 * u, N // 16, v_body, (bvs, bis))

    for r in range(8):
        stage[r, pl.ds(0, 16)] = bvs[r]
        stagei[r, pl.ds(0, 16)] = bis[r]
    pltpu.sync_copy(stage, cand_hbm.at[wid])
    pltpu.sync_copy(stagei, candi_hbm.at[wid])


_scan = functools.partial(
    pl.kernel,
    out_type=(jax.ShapeDtypeStruct((NW, 8, 128), jnp.float32),
              jax.ShapeDtypeStruct((NW, 8, 128), jnp.int32)),
    mesh=plsc.VectorSubcoreMesh(core_axis_name="c", subcore_axis_name="s"),
    compiler_params=pltpu.CompilerParams(use_tc_tiling_on_sc=True),
    scratch_types=[
        pltpu.VMEM((N,), jnp.float32),
        pltpu.VMEM((2, 8, N), jnp.float32),
        pltpu.VMEM((8, 128), jnp.float32),
        pltpu.VMEM((8, 128), jnp.int32),
        pltpu.VMEM((16, 8, 128), jnp.float32),
        pltpu.VMEM((128,), jnp.float32),
        pltpu.VMEM_SHARED((N,), jnp.float32),
        pltpu.SemaphoreType.DMA,
        pltpu.SemaphoreType.DMA,
        pltpu.SemaphoreType.DMA,
    ],
)(_scan_body)


def _merge_body(cand_ref, candi_ref, g_ref, out_ref, cbuf, abuf, bbuf, msem):
    v = cand_ref[:, :, :16]
    ix = candi_ref[:, :, :16]
    m = jnp.min(v)
    win = jnp.min(jnp.where(v == m, ix, BIG_I32))
    i_min = win >> 11
    j_min = win & (N - 1)
    ja = pl.multiple_of((j_min >> 7) * 128, 128)
    ia = pl.multiple_of((i_min >> 7) * 128, 128)
    cp = pltpu.make_async_copy(
        g_ref.at[pl.ds(i_min, 1), pl.ds(ja, 128)], cbuf, msem)
    ap = pltpu.make_async_copy(
        g_ref.at[pl.ds(i_min, 1), pl.ds(ia, 128)], abuf, msem)
    bp = pltpu.make_async_copy(
        g_ref.at[pl.ds(j_min, 1), pl.ds(ja, 128)], bbuf, msem)
    cp.start()
    ap.start()
    bp.start()
    cp.wait()
    ap.wait()
    bp.wait()
    l8 = lax.broadcasted_iota(jnp.int32, (1, 128), 1)
    c = jnp.sum(jnp.where(l8 == (j_min & 127), cbuf[...], 0.0))
    a = jnp.sum(jnp.where(l8 == (i_min & 127), abuf[...], 0.0))
    b = jnp.sum(jnp.where(l8 == (j_min & 127), bbuf[...], 0.0))
    gw = (b - c) / (a + b - 2.0 * c + 1e-8)
    gw = jnp.where(c < b, gw, 0.0)
    gw = jnp.where(c < a, gw, 1.0)
    flat = (lax.broadcasted_iota(jnp.int32, (16, 128), 0) * 128
            + lax.broadcasted_iota(jnp.int32, (16, 128), 1))
    out_ref[...] = (jnp.where(flat == i_min, gw, 0.0)
                    + jnp.where(flat == j_min, 1.0 - gw, 0.0))


_merge = pl.pallas_call(
    _merge_body,
    in_specs=[
        pl.BlockSpec((NW, 8, 128), lambda: (0, 0, 0)),
        pl.BlockSpec((NW, 8, 128), lambda: (0, 0, 0)),
        pl.BlockSpec(memory_space=pl.ANY),
    ],
    out_shape=jax.ShapeDtypeStruct((16, 128), jnp.float32),
    scratch_shapes=[
        pltpu.VMEM((1, 128), jnp.float32),
        pltpu.VMEM((1, 128), jnp.float32),
        pltpu.VMEM((1, 128), jnp.float32),
        pltpu.SemaphoreType.DMA,
    ],
)


def kernel(grammian):
    cand, candi = _scan(grammian)
    return _merge(cand, candi, grammian).reshape(N)
